# Initial kernel scaffold; baseline (speedup 1.0000x reference)
import jax
import jax.numpy as jnp
from jax.experimental import pallas as pl


def kernel(x, edge_index, batch, Wl1, bl1, Wr1, Wl2, bl2, Wr2, Wl3, bl3, Wr3, W_fc1, b_fc1, W_fc2, b_fc2, W_cls, b_cls):
    raise NotImplementedError("write your pallas kernel here")


# speedup vs baseline: 5.6348x; 5.6348x over previous
import functools

import jax
import jax.numpy as jnp
from jax import lax
from jax.experimental import pallas as pl
from jax.experimental.pallas import tpu as pltpu
from jax.experimental.pallas import tpu_sc as plsc

N = 100000
E = 1600000
G = 128
L = 16
NS = 16
NC = 2
EPT = 100352
E_PAD = EPT * NS
NSLAB = 100352
ZROWS = NSLAB // NS
WROWS = N // NS
BR = 8
NIT = EPT // (BR * 128)
BN = 2000


def _make_sc_agg(C):
  mesh = plsc.VectorSubcoreMesh(core_axis_name="c", subcore_axis_name="s")
  cpc = C // NC

  @functools.partial(
      pl.kernel,
      out_type=jax.ShapeDtypeStruct((N, C * L), jnp.float32),
      mesh=mesh,
      scratch_types=[
          pltpu.VMEM_SHARED((NSLAB, L), jnp.float32),
          pltpu.VMEM((BR, 128), jnp.int32),
          pltpu.VMEM((BR, 128), jnp.int32),
          pltpu.VMEM((BR, 128), jnp.int32),
          pltpu.VMEM((BR, 128, L), jnp.float32),
          pltpu.VMEM((128, L), jnp.float32),
          pltpu.SemaphoreType.DMA,
          pltpu.SemaphoreType.DMA,
      ],
      compiler_params=pltpu.CompilerParams(use_tc_tiling_on_sc=False),
  )
  def agg_kernel(xflat, srcp, dstp, out, slab, sraw, sidx, didx, rows, zbuf,
                 gsem, ssem):
    cid = lax.axis_index("c")
    sid = lax.axis_index("s")

    def zb(i, carry):
      zbuf[i, :] = jnp.zeros((L,), jnp.float32)
      return carry

    lax.fori_loop(0, 128, zb, 0)

    for cc in range(cpc):
      c = cc * NC + cid
      cbase = c * N

      def zl(z, carry):
        pltpu.sync_copy(zbuf.at[:, :],
                        slab.at[pl.ds(sid * ZROWS + z * 128, 128)])
        return carry

      lax.fori_loop(0, ZROWS // 128, zl, 0)
      plsc.subcore_barrier()

      def ebody(it, carry):
        rb = sid * (EPT // 128) + it * BR
        pltpu.sync_copy(srcp.at[pl.ds(rb, BR)], sraw)
        pltpu.sync_copy(dstp.at[pl.ds(rb, BR)], didx)
        for j in range(BR):
          for q in range(128 // L):
            sidx[j, pl.ds(q * L, L)] = sraw[j, pl.ds(q * L, L)] + cbase
        gs = [pltpu.async_copy(xflat.at[sidx.at[j]], rows.at[j], gsem)
              for j in range(BR)]
        for g_ in gs:
          g_.wait()
        ss = [pltpu.async_copy(rows.at[j], slab.at[didx.at[j]], ssem,
                               add=True)
              for j in range(BR)]
        for s_ in ss:
          s_.wait()
        return carry

      lax.fori_loop(0, NIT, ebody, 0)
      plsc.subcore_barrier()

      pltpu.sync_copy(
          slab.at[pl.ds(sid * WROWS, WROWS)],
          out.at[pl.ds(sid * WROWS, WROWS), pl.ds(c * L, L)])
      plsc.subcore_barrier()

  return agg_kernel


def _layer1_tc(agg, x, wlt, wrt, b):
  def body(agg_ref, x_ref, wl_ref, wr_ref, b_ref, out_ref, invc_ref):
    a = agg_ref[...]
    cnt = a[:, 50:51]
    invc = 1.0 / jnp.maximum(cnt, 1.0)
    mean = a * invc
    h = (jnp.dot(mean, wl_ref[...], preferred_element_type=jnp.float32)
         + jnp.dot(x_ref[...], wr_ref[...],
                   preferred_element_type=jnp.float32)
         + b_ref[...])
    out_ref[...] = jnp.where(h > 0, h, 0.01 * h)
    invc_ref[...] = invc

  dp, dout = wlt.shape
  return pl.pallas_call(
      body,
      grid=(N // BN,),
      in_specs=[
          pl.BlockSpec((BN, dp), lambda i: (i, 0)),
          pl.BlockSpec((BN, dp), lambda i: (i, 0)),
          pl.BlockSpec((dp, dout), lambda i: (0, 0)),
          pl.BlockSpec((dp, dout), lambda i: (0, 0)),
          pl.BlockSpec((1, dout), lambda i: (0, 0)),
      ],
      out_specs=[
          pl.BlockSpec((BN, dout), lambda i: (i, 0)),
          pl.BlockSpec((BN, 1), lambda i: (i, 0)),
      ],
      out_shape=[
          jax.ShapeDtypeStruct((N, dout), jnp.float32),
          jax.ShapeDtypeStruct((N, 1), jnp.float32),
      ],
  )(agg, x, wlt, wrt, b)


def _layer_tc(agg, x, invc, wlt, wrt, b):
  def body(agg_ref, x_ref, invc_ref, wl_ref, wr_ref, b_ref, out_ref):
    mean = agg_ref[...] * invc_ref[...]
    h = (jnp.dot(mean, wl_ref[...], preferred_element_type=jnp.float32)
         + jnp.dot(x_ref[...], wr_ref[...],
                   preferred_element_type=jnp.float32)
         + b_ref[...])
    out_ref[...] = jnp.maximum(h, 0.0)

  dp, dout = wlt.shape
  dx = x.shape[1]
  return pl.pallas_call(
      body,
      grid=(N // BN,),
      in_specs=[
          pl.BlockSpec((BN, dp), lambda i: (i, 0)),
          pl.BlockSpec((BN, dx), lambda i: (i, 0)),
          pl.BlockSpec((BN, 1), lambda i: (i, 0)),
          pl.BlockSpec((dp, dout), lambda i: (0, 0)),
          pl.BlockSpec((dp, dout), lambda i: (0, 0)),
          pl.BlockSpec((1, dout), lambda i: (0, 0)),
      ],
      out_specs=pl.BlockSpec((BN, dout), lambda i: (i, 0)),
      out_shape=jax.ShapeDtypeStruct((N, dout), jnp.float32),
  )(agg, x, invc, wlt, wrt, b)


def _pool_mlp_tc(x3, batch3, w1, b1, w2, b2, wc, bc):
  d3 = x3.shape[1]
  nsteps = N // BN

  def body(x_ref, b_ref, w1_ref, b1_ref, w2_ref, b2_ref, wc_ref, bc_ref,
           out_ref, psum, pcnt):
    i = pl.program_id(0)

    @pl.when(i == 0)
    def _init():
      psum[...] = jnp.zeros_like(psum)
      pcnt[...] = jnp.zeros_like(pcnt)

    ids = b_ref[0, 0, :]
    oh = (lax.broadcasted_iota(jnp.int32, (G, BN), 0)
          == ids[None, :]).astype(jnp.float32)
    psum[...] += jnp.dot(oh, x_ref[...], preferred_element_type=jnp.float32)
    pcnt[...] += jnp.sum(oh, axis=1, keepdims=True)

    @pl.when(i == nsteps - 1)
    def _final():
      inv = 1.0 / jnp.maximum(pcnt[...], 1.0)
      xp = psum[...] * inv
      x4 = jnp.maximum(
          jnp.dot(xp, w1_ref[...], preferred_element_type=jnp.float32)
          + b1_ref[...], 0.0)
      x5 = jnp.maximum(
          jnp.dot(x4, w2_ref[...], preferred_element_type=jnp.float32)
          + b2_ref[...], 0.0)
      out_ref[...] = (jnp.dot(x5, wc_ref[...],
                              preferred_element_type=jnp.float32)
                      + bc_ref[...])

  return pl.pallas_call(
      body,
      grid=(nsteps,),
      in_specs=[
          pl.BlockSpec((BN, d3), lambda i: (i, 0)),
          pl.BlockSpec((1, 1, BN), lambda i: (i, 0, 0)),
          pl.BlockSpec(w1.shape, lambda i: (0, 0)),
          pl.BlockSpec(b1.shape, lambda i: (0, 0)),
          pl.BlockSpec(w2.shape, lambda i: (0, 0)),
          pl.BlockSpec(b2.shape, lambda i: (0, 0)),
          pl.BlockSpec(wc.shape, lambda i: (0, 0)),
          pl.BlockSpec(bc.shape, lambda i: (0, 0)),
      ],
      out_specs=pl.BlockSpec((G, 128), lambda i: (0, 0)),
      out_shape=jax.ShapeDtypeStruct((G, 128), jnp.float32),
      scratch_shapes=[
          pltpu.VMEM((G, d3), jnp.float32),
          pltpu.VMEM((G, 1), jnp.float32),
      ],
  )(x3, batch3, w1, b1, w2, b2, wc, bc)


def _chunk(h, c):
  return h.reshape(N, c, L).transpose(1, 0, 2).reshape(c * N, L)


def kernel(x, edge_index, batch, Wl1, bl1, Wr1, Wl2, bl2, Wr2, Wl3, bl3, Wr3,
           W_fc1, b_fc1, W_fc2, b_fc2, W_cls, b_cls):
  f32 = jnp.float32
  src = edge_index[0]
  dst = edge_index[1]
  npad = E_PAD - E
  srcp = jnp.concatenate([src, jnp.zeros((npad,), jnp.int32)]).reshape(-1, 128)
  dstp = jnp.concatenate(
      [dst, jnp.full((npad,), N, jnp.int32)]).reshape(-1, 128)

  x0p = jnp.concatenate(
      [x, jnp.ones((N, 1), f32), jnp.zeros((N, 13), f32)], axis=1)

  wl1t = jnp.pad(Wl1.T, ((0, 14), (0, 0)))
  wr1t = jnp.pad(Wr1.T, ((0, 14), (0, 0)))
  wl2t, wr2t = Wl2.T, Wr2.T
  wl3t, wr3t = Wl3.T, Wr3.T
  wct = jnp.pad(W_cls.T, ((0, 0), (0, 128 - 15)))
  bct = jnp.pad(b_cls[None, :], ((0, 0), (0, 128 - 15)))

  agg_fn4 = _make_sc_agg(4)
  agg_fn8 = _make_sc_agg(8)

  agg1 = agg_fn4(_chunk(x0p, 4), srcp, dstp)
  h1, invc = _layer1_tc(agg1, x0p, wl1t, wr1t, bl1[None, :])

  agg2 = agg_fn4(_chunk(h1, 4), srcp, dstp)
  h2 = _layer_tc(agg2, h1, invc, wl2t, wr2t, bl2[None, :])

  agg3 = agg_fn8(_chunk(h2, 8), srcp, dstp)
  h3 = _layer_tc(agg3, h2, invc, wl3t, wr3t, bl3[None, :])

  batch3 = batch.reshape(N // BN, 1, BN)
  outp = _pool_mlp_tc(h3, batch3, W_fc1.T, b_fc1[None, :], W_fc2.T,
                      b_fc2[None, :], wct, bct)
  return outp[:, :15]


# speedup vs baseline: 6.7097x; 1.1908x over previous
import functools

import jax
import jax.numpy as jnp
from jax import lax
from jax.experimental import pallas as pl
from jax.experimental.pallas import tpu as pltpu
from jax.experimental.pallas import tpu_sc as plsc

N = 100000
E = 1600000
G = 128
L = 16
NS = 16
NC = 2
EPT = 100352
E_PAD = EPT * NS
NSLAB = 100352
ZROWS = NSLAB // NS
WROWS = N // NS
BR = 4
NIT = EPT // (BR * 128)
BN = 2000


def _make_sc_agg(C):
  mesh = plsc.VectorSubcoreMesh(core_axis_name="c", subcore_axis_name="s")
  cpc = C // NC

  @functools.partial(
      pl.kernel,
      out_type=jax.ShapeDtypeStruct((N, C * L), jnp.float32),
      mesh=mesh,
      scratch_types=[
          pltpu.VMEM_SHARED((NSLAB, L), jnp.float32),
          pltpu.VMEM((2, BR, 128), jnp.int32),
          pltpu.VMEM((2, BR, 128), jnp.int32),
          pltpu.VMEM((2, BR, 128, L), jnp.float32),
          pltpu.VMEM((128, L), jnp.float32),
          pltpu.SemaphoreType.DMA,
          pltpu.SemaphoreType.DMA,
          pltpu.SemaphoreType.DMA,
          pltpu.SemaphoreType.DMA,
      ],
      compiler_params=pltpu.CompilerParams(use_tc_tiling_on_sc=False),
  )
  def agg_kernel(xflat, srcp, dstp, out, slab, sidx, didx, rows, zbuf,
                 gsem0, gsem1, ssem0, ssem1):
    cid = lax.axis_index("c")
    sid = lax.axis_index("s")
    gsems = (gsem0, gsem1)
    ssems = (ssem0, ssem1)

    def zb(i, carry):
      zbuf[i, :] = jnp.zeros((L,), jnp.float32)
      return carry

    lax.fori_loop(0, 128, zb, 0)

    def load_idx(s, b):
      rb = sid * (EPT // 128) + b * BR
      pltpu.sync_copy(srcp.at[pl.ds(rb, BR)], sidx.at[s])
      pltpu.sync_copy(dstp.at[pl.ds(rb, BR)], didx.at[s])

    def shift_idx(s, cbase):
      for j in range(BR):
        for q in range(128 // L):
          sidx[s, j, pl.ds(q * L, L)] = sidx[s, j, pl.ds(q * L, L)] + cbase

    def fire_g(s):
      for j in range(BR):
        pltpu.async_copy(xflat.at[sidx.at[s, j]], rows.at[s, j], gsems[s])

    def wait_g(s):
      for j in range(BR):
        pltpu.make_async_copy(xflat.at[sidx.at[s, j]], rows.at[s, j],
                              gsems[s]).wait()

    def fire_s(s):
      for j in range(BR):
        pltpu.async_copy(rows.at[s, j], slab.at[didx.at[s, j]], ssems[s],
                         add=True)

    def wait_s(s):
      for j in range(BR):
        pltpu.make_async_copy(rows.at[s, j], slab.at[didx.at[s, j]],
                              ssems[s]).wait()

    for cc in range(cpc):
      c = cc * NC + cid
      cbase = c * N

      for z in range(ZROWS // 128):
        pltpu.async_copy(zbuf.at[:, :],
                         slab.at[pl.ds(sid * ZROWS + z * 128, 128)], gsem0)
      for z in range(ZROWS // 128):
        pltpu.make_async_copy(zbuf.at[:, :],
                              slab.at[pl.ds(sid * ZROWS + z * 128, 128)],
                              gsem0).wait()
      plsc.subcore_barrier()

      load_idx(0, 0)
      shift_idx(0, cbase)
      fire_g(0)

      def ebody(k, carry):
        @pl.when(k > 0)
        def _():
          wait_s(1)
        load_idx(1, 2 * k + 1)
        shift_idx(1, cbase)
        wait_g(0)
        fire_s(0)
        fire_g(1)
        @pl.when(k < NIT // 2 - 1)
        def _():
          load_idx(0, 2 * k + 2)
          shift_idx(0, cbase)
        wait_s(0)
        wait_g(1)
        fire_s(1)
        @pl.when(k < NIT // 2 - 1)
        def _():
          fire_g(0)
        return carry

      lax.fori_loop(0, NIT // 2, ebody, 0)
      wait_s(1)
      plsc.subcore_barrier()

      pltpu.sync_copy(
          slab.at[pl.ds(sid * WROWS, WROWS)],
          out.at[pl.ds(sid * WROWS, WROWS), pl.ds(c * L, L)])
      plsc.subcore_barrier()

  return agg_kernel


def _layer1_tc(agg, x, wlt, wrt, b):
  def body(agg_ref, x_ref, wl_ref, wr_ref, b_ref, out_ref, invc_ref):
    a = agg_ref[...]
    cnt = a[:, 50:51]
    invc = 1.0 / jnp.maximum(cnt, 1.0)
    mean = a * invc
    h = (jnp.dot(mean, wl_ref[...], preferred_element_type=jnp.float32)
         + jnp.dot(x_ref[...], wr_ref[...],
                   preferred_element_type=jnp.float32)
         + b_ref[...])
    out_ref[...] = jnp.where(h > 0, h, 0.01 * h)
    invc_ref[...] = invc

  dp, dout = wlt.shape
  return pl.pallas_call(
      body,
      grid=(N // BN,),
      in_specs=[
          pl.BlockSpec((BN, dp), lambda i: (i, 0)),
          pl.BlockSpec((BN, dp), lambda i: (i, 0)),
          pl.BlockSpec((dp, dout), lambda i: (0, 0)),
          pl.BlockSpec((dp, dout), lambda i: (0, 0)),
          pl.BlockSpec((1, dout), lambda i: (0, 0)),
      ],
      out_specs=[
          pl.BlockSpec((BN, dout), lambda i: (i, 0)),
          pl.BlockSpec((BN, 1), lambda i: (i, 0)),
      ],
      out_shape=[
          jax.ShapeDtypeStruct((N, dout), jnp.float32),
          jax.ShapeDtypeStruct((N, 1), jnp.float32),
      ],
  )(agg, x, wlt, wrt, b)


def _layer_tc(agg, x, invc, wlt, wrt, b):
  def body(agg_ref, x_ref, invc_ref, wl_ref, wr_ref, b_ref, out_ref):
    mean = agg_ref[...] * invc_ref[...]
    h = (jnp.dot(mean, wl_ref[...], preferred_element_type=jnp.float32)
         + jnp.dot(x_ref[...], wr_ref[...],
                   preferred_element_type=jnp.float32)
         + b_ref[...])
    out_ref[...] = jnp.maximum(h, 0.0)

  dp, dout = wlt.shape
  dx = x.shape[1]
  return pl.pallas_call(
      body,
      grid=(N // BN,),
      in_specs=[
          pl.BlockSpec((BN, dp), lambda i: (i, 0)),
          pl.BlockSpec((BN, dx), lambda i: (i, 0)),
          pl.BlockSpec((BN, 1), lambda i: (i, 0)),
          pl.BlockSpec((dp, dout), lambda i: (0, 0)),
          pl.BlockSpec((dp, dout), lambda i: (0, 0)),
          pl.BlockSpec((1, dout), lambda i: (0, 0)),
      ],
      out_specs=pl.BlockSpec((BN, dout), lambda i: (i, 0)),
      out_shape=jax.ShapeDtypeStruct((N, dout), jnp.float32),
  )(agg, x, invc, wlt, wrt, b)


def _pool_mlp_tc(x3, batch3, w1, b1, w2, b2, wc, bc):
  d3 = x3.shape[1]
  nsteps = N // BN

  def body(x_ref, b_ref, w1_ref, b1_ref, w2_ref, b2_ref, wc_ref, bc_ref,
           out_ref, psum, pcnt):
    i = pl.program_id(0)

    @pl.when(i == 0)
    def _init():
      psum[...] = jnp.zeros_like(psum)
      pcnt[...] = jnp.zeros_like(pcnt)

    ids = b_ref[0, 0, :]
    oh = (lax.broadcasted_iota(jnp.int32, (G, BN), 0)
          == ids[None, :]).astype(jnp.float32)
    psum[...] += jnp.dot(oh, x_ref[...], preferred_element_type=jnp.float32)
    pcnt[...] += jnp.sum(oh, axis=1, keepdims=True)

    @pl.when(i == nsteps - 1)
    def _final():
      inv = 1.0 / jnp.maximum(pcnt[...], 1.0)
      xp = psum[...] * inv
      x4 = jnp.maximum(
          jnp.dot(xp, w1_ref[...], preferred_element_type=jnp.float32)
          + b1_ref[...], 0.0)
      x5 = jnp.maximum(
          jnp.dot(x4, w2_ref[...], preferred_element_type=jnp.float32)
          + b2_ref[...], 0.0)
      out_ref[...] = (jnp.dot(x5, wc_ref[...],
                              preferred_element_type=jnp.float32)
                      + bc_ref[...])

  return pl.pallas_call(
      body,
      grid=(nsteps,),
      in_specs=[
          pl.BlockSpec((BN, d3), lambda i: (i, 0)),
          pl.BlockSpec((1, 1, BN), lambda i: (i, 0, 0)),
          pl.BlockSpec(w1.shape, lambda i: (0, 0)),
          pl.BlockSpec(b1.shape, lambda i: (0, 0)),
          pl.BlockSpec(w2.shape, lambda i: (0, 0)),
          pl.BlockSpec(b2.shape, lambda i: (0, 0)),
          pl.BlockSpec(wc.shape, lambda i: (0, 0)),
          pl.BlockSpec(bc.shape, lambda i: (0, 0)),
      ],
      out_specs=pl.BlockSpec((G, 128), lambda i: (0, 0)),
      out_shape=jax.ShapeDtypeStruct((G, 128), jnp.float32),
      scratch_shapes=[
          pltpu.VMEM((G, d3), jnp.float32),
          pltpu.VMEM((G, 1), jnp.float32),
      ],
  )(x3, batch3, w1, b1, w2, b2, wc, bc)


def _chunk(h, c):
  return h.reshape(N, c, L).transpose(1, 0, 2).reshape(c * N, L)


def kernel(x, edge_index, batch, Wl1, bl1, Wr1, Wl2, bl2, Wr2, Wl3, bl3, Wr3,
           W_fc1, b_fc1, W_fc2, b_fc2, W_cls, b_cls):
  f32 = jnp.float32
  src = edge_index[0]
  dst = edge_index[1]
  npad = E_PAD - E
  srcp = jnp.concatenate([src, jnp.zeros((npad,), jnp.int32)]).reshape(-1, 128)
  dstp = jnp.concatenate(
      [dst, jnp.full((npad,), N, jnp.int32)]).reshape(-1, 128)

  x0p = jnp.concatenate(
      [x, jnp.ones((N, 1), f32), jnp.zeros((N, 13), f32)], axis=1)

  wl1t = jnp.pad(Wl1.T, ((0, 14), (0, 0)))
  wr1t = jnp.pad(Wr1.T, ((0, 14), (0, 0)))
  wl2t, wr2t = Wl2.T, Wr2.T
  wl3t, wr3t = Wl3.T, Wr3.T
  wct = jnp.pad(W_cls.T, ((0, 0), (0, 128 - 15)))
  bct = jnp.pad(b_cls[None, :], ((0, 0), (0, 128 - 15)))

  agg_fn4 = _make_sc_agg(4)
  agg_fn8 = _make_sc_agg(8)

  agg1 = agg_fn4(_chunk(x0p, 4), srcp, dstp)
  h1, invc = _layer1_tc(agg1, x0p, wl1t, wr1t, bl1[None, :])

  agg2 = agg_fn4(_chunk(h1, 4), srcp, dstp)
  h2 = _layer_tc(agg2, h1, invc, wl2t, wr2t, bl2[None, :])

  agg3 = agg_fn8(_chunk(h2, 8), srcp, dstp)
  h3 = _layer_tc(agg3, h2, invc, wl3t, wr3t, bl3[None, :])

  batch3 = batch.reshape(N // BN, 1, BN)
  outp = _pool_mlp_tc(h3, batch3, W_fc1.T, b_fc1[None, :], W_fc2.T,
                      b_fc2[None, :], wct, bct)
  return outp[:, :15]


# speedup vs baseline: 7.7176x; 1.1502x over previous
import functools

import jax
import jax.numpy as jnp
from jax import lax
from jax.experimental import pallas as pl
from jax.experimental.pallas import tpu as pltpu
from jax.experimental.pallas import tpu_sc as plsc

N = 100000
E = 1600000
G = 128
L = 16
NS = 16
NC = 2
EPT = 100352
E_PAD = EPT * NS
NSLAB = 100352
ZROWS = NSLAB // NS
WROWS = N // NS
BR = 4
NIT = EPT // (BR * 128)
BN = 2000


def _make_sc_agg(C):
  mesh = plsc.VectorSubcoreMesh(core_axis_name="c", subcore_axis_name="s")
  cpc = C // NC

  @functools.partial(
      pl.kernel,
      out_type=jax.ShapeDtypeStruct((N, C * L), jnp.float32),
      mesh=mesh,
      scratch_types=[
          pltpu.VMEM_SHARED((NSLAB, L), jnp.float32),
          pltpu.VMEM((2, BR, 128), jnp.int32),
          pltpu.VMEM((2, BR, 128), jnp.int32),
          pltpu.VMEM((2, BR, 128, L), jnp.float32),
          pltpu.VMEM((128, L), jnp.float32),
          pltpu.SemaphoreType.DMA,
          pltpu.SemaphoreType.DMA,
          pltpu.SemaphoreType.DMA,
          pltpu.SemaphoreType.DMA,
      ],
      compiler_params=pltpu.CompilerParams(use_tc_tiling_on_sc=False),
  )
  def agg_kernel(xflat, srcp, dstp, out, slab, sidx, didx, rows, zbuf,
                 gsem0, gsem1, ssem0, ssem1):
    cid = lax.axis_index("c")
    sid = lax.axis_index("s")
    gsems = (gsem0, gsem1)
    ssems = (ssem0, ssem1)

    def zb(i, carry):
      zbuf[i, :] = jnp.zeros((L,), jnp.float32)
      return carry

    lax.fori_loop(0, 128, zb, 0)

    def load_idx(s, b):
      rb = sid * (EPT // 128) + b * BR
      pltpu.sync_copy(srcp.at[pl.ds(rb, BR)], sidx.at[s])
      pltpu.sync_copy(dstp.at[pl.ds(rb, BR)], didx.at[s])

    def shift_idx(s, c):
      for j in range(BR):
        for q in range(128 // L):
          sidx[s, j, pl.ds(q * L, L)] = sidx[s, j, pl.ds(q * L, L)] * C + c

    def fire_g(s):
      for j in range(BR):
        pltpu.async_copy(xflat.at[sidx.at[s, j]], rows.at[s, j], gsems[s])

    def wait_g(s):
      for j in range(BR):
        pltpu.make_async_copy(xflat.at[sidx.at[s, j]], rows.at[s, j],
                              gsems[s]).wait()

    def fire_s(s):
      for j in range(BR):
        pltpu.async_copy(rows.at[s, j], slab.at[didx.at[s, j]], ssems[s],
                         add=True)

    def wait_s(s):
      for j in range(BR):
        pltpu.make_async_copy(rows.at[s, j], slab.at[didx.at[s, j]],
                              ssems[s]).wait()

    for cc in range(cpc):
      c = cc * NC + cid

      for z in range(ZROWS // 128):
        pltpu.async_copy(zbuf.at[:, :],
                         slab.at[pl.ds(sid * ZROWS + z * 128, 128)], gsem0)
      for z in range(ZROWS // 128):
        pltpu.make_async_copy(zbuf.at[:, :],
                              slab.at[pl.ds(sid * ZROWS + z * 128, 128)],
                              gsem0).wait()
      plsc.subcore_barrier()

      load_idx(0, 0)
      shift_idx(0, c)
      fire_g(0)

      def ebody(k, carry):
        @pl.when(k > 0)
        def _():
          wait_s(1)
        load_idx(1, 2 * k + 1)
        shift_idx(1, c)
        wait_g(0)
        fire_s(0)
        fire_g(1)
        @pl.when(k < NIT // 2 - 1)
        def _():
          load_idx(0, 2 * k + 2)
          shift_idx(0, c)
        wait_s(0)
        wait_g(1)
        fire_s(1)
        @pl.when(k < NIT // 2 - 1)
        def _():
          fire_g(0)
        return carry

      lax.fori_loop(0, NIT // 2, ebody, 0)
      wait_s(1)
      plsc.subcore_barrier()

      pltpu.sync_copy(
          slab.at[pl.ds(sid * WROWS, WROWS)],
          out.at[pl.ds(sid * WROWS, WROWS), pl.ds(c * L, L)])
      plsc.subcore_barrier()

  return agg_kernel


def _layer1_tc(agg, x, wlt, wrt, b):
  def body(agg_ref, x_ref, wl_ref, wr_ref, b_ref, out_ref, invc_ref):
    a = agg_ref[...]
    cnt = a[:, 50:51]
    invc = 1.0 / jnp.maximum(cnt, 1.0)
    mean = a * invc
    h = (jnp.dot(mean, wl_ref[...], preferred_element_type=jnp.float32)
         + jnp.dot(x_ref[...], wr_ref[...],
                   preferred_element_type=jnp.float32)
         + b_ref[...])
    out_ref[...] = jnp.where(h > 0, h, 0.01 * h)
    invc_ref[...] = invc

  dp, dout = wlt.shape
  return pl.pallas_call(
      body,
      grid=(N // BN,),
      in_specs=[
          pl.BlockSpec((BN, dp), lambda i: (i, 0)),
          pl.BlockSpec((BN, dp), lambda i: (i, 0)),
          pl.BlockSpec((dp, dout), lambda i: (0, 0)),
          pl.BlockSpec((dp, dout), lambda i: (0, 0)),
          pl.BlockSpec((1, dout), lambda i: (0, 0)),
      ],
      out_specs=[
          pl.BlockSpec((BN, dout), lambda i: (i, 0)),
          pl.BlockSpec((BN, 1), lambda i: (i, 0)),
      ],
      out_shape=[
          jax.ShapeDtypeStruct((N, dout), jnp.float32),
          jax.ShapeDtypeStruct((N, 1), jnp.float32),
      ],
  )(agg, x, wlt, wrt, b)


def _layer_tc(agg, x, invc, wlt, wrt, b):
  def body(agg_ref, x_ref, invc_ref, wl_ref, wr_ref, b_ref, out_ref):
    mean = agg_ref[...] * invc_ref[...]
    h = (jnp.dot(mean, wl_ref[...], preferred_element_type=jnp.float32)
         + jnp.dot(x_ref[...], wr_ref[...],
                   preferred_element_type=jnp.float32)
         + b_ref[...])
    out_ref[...] = jnp.maximum(h, 0.0)

  dp, dout = wlt.shape
  dx = x.shape[1]
  return pl.pallas_call(
      body,
      grid=(N // BN,),
      in_specs=[
          pl.BlockSpec((BN, dp), lambda i: (i, 0)),
          pl.BlockSpec((BN, dx), lambda i: (i, 0)),
          pl.BlockSpec((BN, 1), lambda i: (i, 0)),
          pl.BlockSpec((dp, dout), lambda i: (0, 0)),
          pl.BlockSpec((dp, dout), lambda i: (0, 0)),
          pl.BlockSpec((1, dout), lambda i: (0, 0)),
      ],
      out_specs=pl.BlockSpec((BN, dout), lambda i: (i, 0)),
      out_shape=jax.ShapeDtypeStruct((N, dout), jnp.float32),
  )(agg, x, invc, wlt, wrt, b)


def _pool_mlp_tc(x3, batch3, w1, b1, w2, b2, wc, bc):
  d3 = x3.shape[1]
  nsteps = N // BN

  def body(x_ref, b_ref, w1_ref, b1_ref, w2_ref, b2_ref, wc_ref, bc_ref,
           out_ref, psum, pcnt):
    i = pl.program_id(0)

    @pl.when(i == 0)
    def _init():
      psum[...] = jnp.zeros_like(psum)
      pcnt[...] = jnp.zeros_like(pcnt)

    ids = b_ref[0, 0, :]
    oh = (lax.broadcasted_iota(jnp.int32, (G, BN), 0)
          == ids[None, :]).astype(jnp.float32)
    psum[...] += jnp.dot(oh, x_ref[...], preferred_element_type=jnp.float32)
    pcnt[...] += jnp.sum(oh, axis=1, keepdims=True)

    @pl.when(i == nsteps - 1)
    def _final():
      inv = 1.0 / jnp.maximum(pcnt[...], 1.0)
      xp = psum[...] * inv
      x4 = jnp.maximum(
          jnp.dot(xp, w1_ref[...], preferred_element_type=jnp.float32)
          + b1_ref[...], 0.0)
      x5 = jnp.maximum(
          jnp.dot(x4, w2_ref[...], preferred_element_type=jnp.float32)
          + b2_ref[...], 0.0)
      out_ref[...] = (jnp.dot(x5, wc_ref[...],
                              preferred_element_type=jnp.float32)
                      + bc_ref[...])

  return pl.pallas_call(
      body,
      grid=(nsteps,),
      in_specs=[
          pl.BlockSpec((BN, d3), lambda i: (i, 0)),
          pl.BlockSpec((1, 1, BN), lambda i: (i, 0, 0)),
          pl.BlockSpec(w1.shape, lambda i: (0, 0)),
          pl.BlockSpec(b1.shape, lambda i: (0, 0)),
          pl.BlockSpec(w2.shape, lambda i: (0, 0)),
          pl.BlockSpec(b2.shape, lambda i: (0, 0)),
          pl.BlockSpec(wc.shape, lambda i: (0, 0)),
          pl.BlockSpec(bc.shape, lambda i: (0, 0)),
      ],
      out_specs=pl.BlockSpec((G, 128), lambda i: (0, 0)),
      out_shape=jax.ShapeDtypeStruct((G, 128), jnp.float32),
      scratch_shapes=[
          pltpu.VMEM((G, d3), jnp.float32),
          pltpu.VMEM((G, 1), jnp.float32),
      ],
  )(x3, batch3, w1, b1, w2, b2, wc, bc)


def kernel(x, edge_index, batch, Wl1, bl1, Wr1, Wl2, bl2, Wr2, Wl3, bl3, Wr3,
           W_fc1, b_fc1, W_fc2, b_fc2, W_cls, b_cls):
  f32 = jnp.float32
  src = edge_index[0]
  dst = edge_index[1]
  npad = E_PAD - E
  srcp = jnp.concatenate([src, jnp.zeros((npad,), jnp.int32)]).reshape(-1, 128)
  dstp = jnp.concatenate(
      [dst, jnp.full((npad,), N, jnp.int32)]).reshape(-1, 128)

  x0p = jnp.concatenate(
      [x, jnp.ones((N, 1), f32), jnp.zeros((N, 13), f32)], axis=1)

  wl1t = jnp.pad(Wl1.T, ((0, 14), (0, 0)))
  wr1t = jnp.pad(Wr1.T, ((0, 14), (0, 0)))
  wl2t, wr2t = Wl2.T, Wr2.T
  wl3t, wr3t = Wl3.T, Wr3.T
  wct = jnp.pad(W_cls.T, ((0, 0), (0, 128 - 15)))
  bct = jnp.pad(b_cls[None, :], ((0, 0), (0, 128 - 15)))

  agg_fn4 = _make_sc_agg(4)
  agg_fn8 = _make_sc_agg(8)

  agg1 = agg_fn4(x0p.reshape(4 * N, L), srcp, dstp)
  h1, invc = _layer1_tc(agg1, x0p, wl1t, wr1t, bl1[None, :])

  agg2 = agg_fn4(h1.reshape(4 * N, L), srcp, dstp)
  h2 = _layer_tc(agg2, h1, invc, wl2t, wr2t, bl2[None, :])

  agg3 = agg_fn8(h2.reshape(8 * N, L), srcp, dstp)
  h3 = _layer_tc(agg3, h2, invc, wl3t, wr3t, bl3[None, :])

  batch3 = batch.reshape(N // BN, 1, BN)
  outp = _pool_mlp_tc(h3, batch3, W_fc1.T, b_fc1[None, :], W_fc2.T,
                      b_fc2[None, :], wct, bct)
  return outp[:, :15]


# speedup vs baseline: 9.7822x; 1.2675x over previous
import functools

import jax
import jax.numpy as jnp
from jax import lax
from jax.experimental import pallas as pl
from jax.experimental.pallas import tpu as pltpu
from jax.experimental.pallas import tpu_sc as plsc

N = 100000
E = 1600000
G = 128
L = 16
NS = 16
NC = 2
EPT = 100352
E_PAD = EPT * NS
NSLAB = 100352
ZROWS = NSLAB // NS
WROWS = N // NS
BR = 4
NIT = EPT // (BR * 128)
BN = 2000


def _make_sc_agg(C):
  mesh = plsc.VectorSubcoreMesh(core_axis_name="c", subcore_axis_name="s")
  cpc = C // NC

  @functools.partial(
      pl.kernel,
      out_type=jax.ShapeDtypeStruct((N, C * L), jnp.float32),
      mesh=mesh,
      scratch_types=[
          pltpu.VMEM_SHARED((NSLAB, L), jnp.float32),
          pltpu.VMEM((4, BR, 128), jnp.int32),
          pltpu.VMEM((4, BR, 128), jnp.int32),
          pltpu.VMEM((2, BR, 128, L), jnp.float32),
          pltpu.VMEM((128, L), jnp.float32),
          pltpu.SemaphoreType.DMA,
          pltpu.SemaphoreType.DMA,
          pltpu.SemaphoreType.DMA,
          pltpu.SemaphoreType.DMA,
          pltpu.SemaphoreType.DMA,
          pltpu.SemaphoreType.DMA,
      ],
      compiler_params=pltpu.CompilerParams(use_tc_tiling_on_sc=False),
  )
  def agg_kernel(xflat, srcp, dstp, out, slab, sidx, didx, rows, zbuf,
                 gsem0, gsem1, ssem0, ssem1, isem0, isem1):
    cid = lax.axis_index("c")
    sid = lax.axis_index("s")
    gsems = (gsem0, gsem1)
    ssems = (ssem0, ssem1)
    isems = (isem0, isem1)

    def zb(i, carry):
      zbuf[i, :] = jnp.zeros((L,), jnp.float32)
      return carry

    lax.fori_loop(0, 128, zb, 0)

    def fire_idx(ii, b, par):
      rb = sid * (EPT // 128) + b * BR
      pltpu.async_copy(srcp.at[pl.ds(rb, BR)], sidx.at[ii], isems[par])
      pltpu.async_copy(dstp.at[pl.ds(rb, BR)], didx.at[ii], isems[par])

    def wait_idx(par):
      pltpu.make_async_copy(srcp.at[pl.ds(0, BR)], sidx.at[0],
                            isems[par]).wait()
      pltpu.make_async_copy(dstp.at[pl.ds(0, BR)], didx.at[0],
                            isems[par]).wait()

    def shift_idx(ii, c):
      for j in range(BR):
        for q in range(128 // L):
          sidx[ii, j, pl.ds(q * L, L)] = sidx[ii, j, pl.ds(q * L, L)] * C + c

    def fire_g(rs, ii):
      for j in range(BR):
        pltpu.async_copy(xflat.at[sidx.at[ii, j]], rows.at[rs, j], gsems[rs])

    def wait_g(rs):
      for j in range(BR):
        pltpu.make_async_copy(xflat.at[sidx.at[0, j]], rows.at[rs, j],
                              gsems[rs]).wait()

    def fire_s(rs, ii):
      for j in range(BR):
        pltpu.async_copy(rows.at[rs, j], slab.at[didx.at[ii, j]], ssems[rs],
                         add=True)

    def wait_s(rs):
      for j in range(BR):
        pltpu.make_async_copy(rows.at[rs, j], slab.at[didx.at[0, j]],
                              ssems[rs]).wait()

    for cc in range(cpc):
      c = cc * NC + cid

      for z in range(ZROWS // 128):
        pltpu.async_copy(zbuf.at[:, :],
                         slab.at[pl.ds(sid * ZROWS + z * 128, 128)], gsem0)
      for z in range(ZROWS // 128):
        pltpu.make_async_copy(zbuf.at[:, :],
                              slab.at[pl.ds(sid * ZROWS + z * 128, 128)],
                              gsem0).wait()
      plsc.subcore_barrier()

      def phase(b, p, do_a, do_b, do_ef):
        rs = p & 1
        ii = p & 3
        if do_a:
          wait_s(rs)
        wait_idx(rs)
        shift_idx(ii, c)
        if do_b:
          fire_idx((ii + 2) & 3, b + 2, rs)
        fire_g(rs, ii)
        if do_ef:
          wait_g(rs ^ 1)
          fire_s(rs ^ 1, (ii + 3) & 3)

      fire_idx(0, 0, 0)
      fire_idx(1, 1, 1)
      phase(0, 0, False, True, False)
      phase(1, 1, False, True, True)

      def ebody(m, carry):
        for p in range(4):
          phase(4 * m + 2 + p, 2 + p, True, True, True)
        return carry

      lax.fori_loop(0, (NIT - 2) // 4, ebody, 0)
      phase(NIT - 2, 2, True, False, True)
      phase(NIT - 1, 3, True, False, True)
      wait_g(1)
      fire_s(1, 3)
      wait_s(0)
      wait_s(1)
      plsc.subcore_barrier()

      pltpu.sync_copy(
          slab.at[pl.ds(sid * WROWS, WROWS)],
          out.at[pl.ds(sid * WROWS, WROWS), pl.ds(c * L, L)])
      plsc.subcore_barrier()

  return agg_kernel


def _layer1_tc(agg, x, wlt, wrt, b):
  def body(agg_ref, x_ref, wl_ref, wr_ref, b_ref, out_ref, invc_ref):
    a = agg_ref[...]
    cnt = a[:, 50:51]
    invc = 1.0 / jnp.maximum(cnt, 1.0)
    mean = a * invc
    h = (jnp.dot(mean, wl_ref[...], preferred_element_type=jnp.float32)
         + jnp.dot(x_ref[...], wr_ref[...],
                   preferred_element_type=jnp.float32)
         + b_ref[...])
    out_ref[...] = jnp.where(h > 0, h, 0.01 * h)
    invc_ref[...] = invc

  dp, dout = wlt.shape
  return pl.pallas_call(
      body,
      grid=(N // BN,),
      in_specs=[
          pl.BlockSpec((BN, dp), lambda i: (i, 0)),
          pl.BlockSpec((BN, dp), lambda i: (i, 0)),
          pl.BlockSpec((dp, dout), lambda i: (0, 0)),
          pl.BlockSpec((dp, dout), lambda i: (0, 0)),
          pl.BlockSpec((1, dout), lambda i: (0, 0)),
      ],
      out_specs=[
          pl.BlockSpec((BN, dout), lambda i: (i, 0)),
          pl.BlockSpec((BN, 1), lambda i: (i, 0)),
      ],
      out_shape=[
          jax.ShapeDtypeStruct((N, dout), jnp.float32),
          jax.ShapeDtypeStruct((N, 1), jnp.float32),
      ],
  )(agg, x, wlt, wrt, b)


def _layer_tc(agg, x, invc, wlt, wrt, b):
  def body(agg_ref, x_ref, invc_ref, wl_ref, wr_ref, b_ref, out_ref):
    mean = agg_ref[...] * invc_ref[...]
    h = (jnp.dot(mean, wl_ref[...], preferred_element_type=jnp.float32)
         + jnp.dot(x_ref[...], wr_ref[...],
                   preferred_element_type=jnp.float32)
         + b_ref[...])
    out_ref[...] = jnp.maximum(h, 0.0)

  dp, dout = wlt.shape
  dx = x.shape[1]
  return pl.pallas_call(
      body,
      grid=(N // BN,),
      in_specs=[
          pl.BlockSpec((BN, dp), lambda i: (i, 0)),
          pl.BlockSpec((BN, dx), lambda i: (i, 0)),
          pl.BlockSpec((BN, 1), lambda i: (i, 0)),
          pl.BlockSpec((dp, dout), lambda i: (0, 0)),
          pl.BlockSpec((dp, dout), lambda i: (0, 0)),
          pl.BlockSpec((1, dout), lambda i: (0, 0)),
      ],
      out_specs=pl.BlockSpec((BN, dout), lambda i: (i, 0)),
      out_shape=jax.ShapeDtypeStruct((N, dout), jnp.float32),
  )(agg, x, invc, wlt, wrt, b)


def _pool_mlp_tc(x3, batch3, w1, b1, w2, b2, wc, bc):
  d3 = x3.shape[1]
  nsteps = N // BN

  def body(x_ref, b_ref, w1_ref, b1_ref, w2_ref, b2_ref, wc_ref, bc_ref,
           out_ref, psum, pcnt):
    i = pl.program_id(0)

    @pl.when(i == 0)
    def _init():
      psum[...] = jnp.zeros_like(psum)
      pcnt[...] = jnp.zeros_like(pcnt)

    ids = b_ref[0, 0, :]
    oh = (lax.broadcasted_iota(jnp.int32, (G, BN), 0)
          == ids[None, :]).astype(jnp.float32)
    psum[...] += jnp.dot(oh, x_ref[...], preferred_element_type=jnp.float32)
    pcnt[...] += jnp.sum(oh, axis=1, keepdims=True)

    @pl.when(i == nsteps - 1)
    def _final():
      inv = 1.0 / jnp.maximum(pcnt[...], 1.0)
      xp = psum[...] * inv
      x4 = jnp.maximum(
          jnp.dot(xp, w1_ref[...], preferred_element_type=jnp.float32)
          + b1_ref[...], 0.0)
      x5 = jnp.maximum(
          jnp.dot(x4, w2_ref[...], preferred_element_type=jnp.float32)
          + b2_ref[...], 0.0)
      out_ref[...] = (jnp.dot(x5, wc_ref[...],
                              preferred_element_type=jnp.float32)
                      + bc_ref[...])

  return pl.pallas_call(
      body,
      grid=(nsteps,),
      in_specs=[
          pl.BlockSpec((BN, d3), lambda i: (i, 0)),
          pl.BlockSpec((1, 1, BN), lambda i: (i, 0, 0)),
          pl.BlockSpec(w1.shape, lambda i: (0, 0)),
          pl.BlockSpec(b1.shape, lambda i: (0, 0)),
          pl.BlockSpec(w2.shape, lambda i: (0, 0)),
          pl.BlockSpec(b2.shape, lambda i: (0, 0)),
          pl.BlockSpec(wc.shape, lambda i: (0, 0)),
          pl.BlockSpec(bc.shape, lambda i: (0, 0)),
      ],
      out_specs=pl.BlockSpec((G, 128), lambda i: (0, 0)),
      out_shape=jax.ShapeDtypeStruct((G, 128), jnp.float32),
      scratch_shapes=[
          pltpu.VMEM((G, d3), jnp.float32),
          pltpu.VMEM((G, 1), jnp.float32),
      ],
  )(x3, batch3, w1, b1, w2, b2, wc, bc)


def kernel(x, edge_index, batch, Wl1, bl1, Wr1, Wl2, bl2, Wr2, Wl3, bl3, Wr3,
           W_fc1, b_fc1, W_fc2, b_fc2, W_cls, b_cls):
  f32 = jnp.float32
  src = edge_index[0]
  dst = edge_index[1]
  npad = E_PAD - E
  srcp = jnp.concatenate([src, jnp.zeros((npad,), jnp.int32)]).reshape(-1, 128)
  dstp = jnp.concatenate(
      [dst, jnp.full((npad,), N, jnp.int32)]).reshape(-1, 128)

  x0p = jnp.concatenate(
      [x, jnp.ones((N, 1), f32), jnp.zeros((N, 13), f32)], axis=1)

  wl1t = jnp.pad(Wl1.T, ((0, 14), (0, 0)))
  wr1t = jnp.pad(Wr1.T, ((0, 14), (0, 0)))
  wl2t, wr2t = Wl2.T, Wr2.T
  wl3t, wr3t = Wl3.T, Wr3.T
  wct = jnp.pad(W_cls.T, ((0, 0), (0, 128 - 15)))
  bct = jnp.pad(b_cls[None, :], ((0, 0), (0, 128 - 15)))

  agg_fn4 = _make_sc_agg(4)
  agg_fn8 = _make_sc_agg(8)

  agg1 = agg_fn4(x0p.reshape(4 * N, L), srcp, dstp)
  h1, invc = _layer1_tc(agg1, x0p, wl1t, wr1t, bl1[None, :])

  agg2 = agg_fn4(h1.reshape(4 * N, L), srcp, dstp)
  h2 = _layer_tc(agg2, h1, invc, wl2t, wr2t, bl2[None, :])

  agg3 = agg_fn8(h2.reshape(8 * N, L), srcp, dstp)
  h3 = _layer_tc(agg3, h2, invc, wl3t, wr3t, bl3[None, :])

  batch3 = batch.reshape(N // BN, 1, BN)
  outp = _pool_mlp_tc(h3, batch3, W_fc1.T, b_fc1[None, :], W_fc2.T,
                      b_fc2[None, :], wct, bct)
  return outp[:, :15]
